# baseline (speedup 1.0000x reference)
import functools
import math

import jax
import jax.numpy as jnp
from jax import lax
from jax.experimental import pallas as pl
from jax.experimental.pallas import tpu as pltpu
from jax.experimental.pallas import tpu_sc as plsc

INPUT_DIM = 2048
NUM_EXPERTS = 16
TOP_K = 2
AUX_COEFF = 0.01

N_TOKENS = 4 * 4096

SC_CORES = 2
SC_SUBCORES = 16
NUM_WORKERS = SC_CORES * SC_SUBCORES
CHUNK = N_TOKENS // NUM_WORKERS
GROUPS = CHUNK // 16

TOK_BLOCK = 512
NUM_BLOCKS = N_TOKENS // TOK_BLOCK

_AUX_CONST = AUX_COEFF * (-math.log(NUM_EXPERTS) / NUM_EXPERTS)
_C_LSE = AUX_COEFF / (NUM_EXPERTS * N_TOKENS)
_C_LOGIT = AUX_COEFF / (NUM_EXPERTS * NUM_EXPERTS * N_TOKENS)


def _tc_body(x_ref, w_ref, b_ref, logits_ref, aux_ref):
    i = pl.program_id(0)
    lg = lax.dot_general(w_ref[...], x_ref[...], (((1,), (1,)), ((), ())),
                         preferred_element_type=jnp.float32)
    lg = lg + b_ref[...]
    logits_ref[...] = lg
    m = jnp.max(lg, axis=0, keepdims=True)
    se = jnp.sum(jnp.exp(lg - m), axis=0, keepdims=True)
    lse_sum = jnp.sum(jnp.log(se) + m)
    logit_sum = jnp.sum(lg)

    @pl.when(i == 0)
    def _():
        aux_ref[0, 0] = _AUX_CONST

    aux_ref[0, 0] += _C_LSE * lse_sum - _C_LOGIT * logit_sum


def _tc_logits_aux(x2d, w, b_col):
    return pl.pallas_call(
        _tc_body,
        grid=(NUM_BLOCKS,),
        in_specs=[
            pl.BlockSpec((TOK_BLOCK, INPUT_DIM), lambda i: (i, 0)),
            pl.BlockSpec((NUM_EXPERTS, INPUT_DIM), lambda i: (0, 0)),
            pl.BlockSpec((NUM_EXPERTS, 1), lambda i: (0, 0)),
        ],
        out_specs=[
            pl.BlockSpec((NUM_EXPERTS, TOK_BLOCK), lambda i: (0, i)),
            pl.BlockSpec((1, 1), lambda i: (0, 0),
                         memory_space=pltpu.SMEM),
        ],
        out_shape=[
            jax.ShapeDtypeStruct((NUM_EXPERTS, N_TOKENS), jnp.float32),
            jax.ShapeDtypeStruct((1, 1), jnp.float32),
        ],
    )(x2d, w, b_col)


def _sc_routing_body(logits_hbm, scores_hbm, idx_hbm, lg_v, sc_v, ix_v):
    wid = lax.axis_index("s") * SC_CORES + lax.axis_index("c")
    base = wid * CHUNK
    pltpu.sync_copy(logits_hbm.at[:, pl.ds(base, CHUNK)], lg_v)

    lane = lax.iota(jnp.int32, 16)

    def group(g, carry):
        offs = g * 16
        m1 = lg_v[0, pl.ds(offs, 16)]
        i1 = jnp.zeros((16,), jnp.int32)
        m2 = jnp.full((16,), -3.0e38, jnp.float32)
        i2 = jnp.zeros((16,), jnp.int32)
        for e in range(1, NUM_EXPERTS):
            v = lg_v[e, pl.ds(offs, 16)]
            gt1 = v > m1
            gt2 = v > m2
            m2 = jnp.where(gt1, m1, jnp.where(gt2, v, m2))
            i2 = jnp.where(gt1, i1, jnp.where(gt2, jnp.int32(e), i2))
            m1 = jnp.where(gt1, v, m1)
            i1 = jnp.where(gt1, jnp.int32(e), i1)
        e1 = jnp.exp(m2 - m1)
        denom = 1.0 + e1
        g0 = 1.0 / denom
        g1 = e1 * g0
        pos0 = lane * 2 + g * 32
        plsc.store_scatter(sc_v, [pos0], g0)
        plsc.store_scatter(sc_v, [pos0 + 1], g1)
        plsc.store_scatter(ix_v, [pos0], i1)
        plsc.store_scatter(ix_v, [pos0 + 1], i2)
        return carry

    lax.fori_loop(0, GROUPS, group, 0)
    pltpu.sync_copy(sc_v, scores_hbm.at[pl.ds(base * 2, 2 * CHUNK)])
    pltpu.sync_copy(ix_v, idx_hbm.at[pl.ds(base * 2, 2 * CHUNK)])


@functools.cache
def _sc_routing():
    return pl.kernel(
        _sc_routing_body,
        out_type=[
            jax.ShapeDtypeStruct((2 * N_TOKENS,), jnp.float32),
            jax.ShapeDtypeStruct((2 * N_TOKENS,), jnp.int32),
        ],
        mesh=plsc.VectorSubcoreMesh(core_axis_name="c", subcore_axis_name="s"),
        compiler_params=pltpu.CompilerParams(needs_layout_passes=False),
        scratch_types=[
            pltpu.VMEM((NUM_EXPERTS, CHUNK), jnp.float32),
            pltpu.VMEM((2 * CHUNK,), jnp.float32),
            pltpu.VMEM((2 * CHUNK,), jnp.int32),
        ],
    )


def kernel(x, W, b):
    B, S, D = x.shape
    x2d = x.reshape(B * S, D)
    logits_t, aux = _tc_logits_aux(x2d, W, b.reshape(NUM_EXPERTS, 1))
    scores_flat, idx_flat = _sc_routing()(logits_t)
    gate_scores = scores_flat.reshape(B, S, TOP_K)
    expert_indices = idx_flat.reshape(B, S, TOP_K)
    return gate_scores, expert_indices, aux[0, 0]


# speedup vs baseline: 1.0941x; 1.0941x over previous
import functools
import math

import jax
import jax.numpy as jnp
from jax import lax
from jax.experimental import pallas as pl
from jax.experimental.pallas import tpu as pltpu
from jax.experimental.pallas import tpu_sc as plsc

INPUT_DIM = 2048
NUM_EXPERTS = 16
TOP_K = 2
AUX_COEFF = 0.01

N_TOKENS = 4 * 4096

SC_CORES = 2
SC_SUBCORES = 16
NUM_WORKERS = SC_CORES * SC_SUBCORES
CHUNK = N_TOKENS // NUM_WORKERS
GROUPS = CHUNK // 16

TOK_BLOCK = 1024
NUM_BLOCKS = N_TOKENS // TOK_BLOCK

_AUX_CONST = AUX_COEFF * (-math.log(NUM_EXPERTS) / NUM_EXPERTS)
_C_LSE = AUX_COEFF / (NUM_EXPERTS * N_TOKENS)
_C_LOGIT = AUX_COEFF / (NUM_EXPERTS * NUM_EXPERTS * N_TOKENS)


def _tc_body(x_ref, w_ref, b_ref, logits_ref, aux_ref):
    i = pl.program_id(0)
    lg = lax.dot_general(w_ref[...], x_ref[...], (((1,), (1,)), ((), ())),
                         preferred_element_type=jnp.float32)
    lg = lg + b_ref[...]
    logits_ref[...] = lg
    m = jnp.max(lg, axis=0, keepdims=True)
    se = jnp.sum(jnp.exp(lg - m), axis=0, keepdims=True)
    lse_sum = jnp.sum(jnp.log(se) + m)
    logit_sum = jnp.sum(lg)

    @pl.when(i == 0)
    def _():
        aux_ref[0, 0] = _AUX_CONST

    aux_ref[0, 0] += _C_LSE * lse_sum - _C_LOGIT * logit_sum


def _tc_logits_aux(x2d, w, b_col):
    return pl.pallas_call(
        _tc_body,
        grid=(NUM_BLOCKS,),
        in_specs=[
            pl.BlockSpec((TOK_BLOCK, INPUT_DIM), lambda i: (i, 0)),
            pl.BlockSpec((NUM_EXPERTS, INPUT_DIM), lambda i: (0, 0)),
            pl.BlockSpec((NUM_EXPERTS, 1), lambda i: (0, 0)),
        ],
        out_specs=[
            pl.BlockSpec((NUM_EXPERTS, TOK_BLOCK), lambda i: (0, i)),
            pl.BlockSpec((1, 1), lambda i: (0, 0),
                         memory_space=pltpu.SMEM),
        ],
        out_shape=[
            jax.ShapeDtypeStruct((NUM_EXPERTS, N_TOKENS), jnp.float32),
            jax.ShapeDtypeStruct((1, 1), jnp.float32),
        ],
    )(x2d, w, b_col)


def _sc_routing_body(logits_hbm, scores_hbm, idx_hbm, lg_v, sc_v, ix_v):
    wid = lax.axis_index("s") * SC_CORES + lax.axis_index("c")
    base = wid * CHUNK
    pltpu.sync_copy(logits_hbm.at[:, pl.ds(base, CHUNK)], lg_v)

    lane = lax.iota(jnp.int32, 16)

    def group(g, carry):
        offs = g * 16
        m1 = lg_v[0, pl.ds(offs, 16)]
        i1 = jnp.zeros((16,), jnp.int32)
        m2 = jnp.full((16,), -3.0e38, jnp.float32)
        i2 = jnp.zeros((16,), jnp.int32)
        for e in range(1, NUM_EXPERTS):
            v = lg_v[e, pl.ds(offs, 16)]
            gt1 = v > m1
            gt2 = v > m2
            m2 = jnp.where(gt1, m1, jnp.where(gt2, v, m2))
            i2 = jnp.where(gt1, i1, jnp.where(gt2, jnp.int32(e), i2))
            m1 = jnp.where(gt1, v, m1)
            i1 = jnp.where(gt1, jnp.int32(e), i1)
        e1 = jnp.exp(m2 - m1)
        denom = 1.0 + e1
        g0 = 1.0 / denom
        g1 = e1 * g0
        pos0 = lane * 2 + g * 32
        plsc.store_scatter(sc_v, [pos0], g0)
        plsc.store_scatter(sc_v, [pos0 + 1], g1)
        plsc.store_scatter(ix_v, [pos0], i1)
        plsc.store_scatter(ix_v, [pos0 + 1], i2)
        return carry

    lax.fori_loop(0, GROUPS, group, 0)
    pltpu.sync_copy(sc_v, scores_hbm.at[pl.ds(base * 2, 2 * CHUNK)])
    pltpu.sync_copy(ix_v, idx_hbm.at[pl.ds(base * 2, 2 * CHUNK)])


@functools.cache
def _sc_routing():
    return pl.kernel(
        _sc_routing_body,
        out_type=[
            jax.ShapeDtypeStruct((2 * N_TOKENS,), jnp.float32),
            jax.ShapeDtypeStruct((2 * N_TOKENS,), jnp.int32),
        ],
        mesh=plsc.VectorSubcoreMesh(core_axis_name="c", subcore_axis_name="s"),
        compiler_params=pltpu.CompilerParams(needs_layout_passes=False),
        scratch_types=[
            pltpu.VMEM((NUM_EXPERTS, CHUNK), jnp.float32),
            pltpu.VMEM((2 * CHUNK,), jnp.float32),
            pltpu.VMEM((2 * CHUNK,), jnp.int32),
        ],
    )


def kernel(x, W, b):
    B, S, D = x.shape
    x2d = x.reshape(B * S, D)
    logits_t, aux = _tc_logits_aux(x2d, W, b.reshape(NUM_EXPERTS, 1))
    scores_flat, idx_flat = _sc_routing()(logits_t)
    gate_scores = scores_flat.reshape(B, S, TOP_K)
    expert_indices = idx_flat.reshape(B, S, TOP_K)
    return gate_scores, expert_indices, aux[0, 0]
